# baseline (speedup 1.0000x reference)
import jax
import jax.numpy as jnp
from jax import lax
from jax.experimental import pallas as pl


_BN = 8
_C = 512


def _loss_body(tgt_ref, x_ref, out_ref):
    v = x_ref.shape[1]
    n_full = v // _C
    tail_st, tail_w = n_full * _C, v % _C

    def chunk(c):
        idx = pl.multiple_of(c * _C, _C)
        return x_ref[:, pl.ds(idx, _C)]

    def max_body(c, accm):
        return jnp.maximum(accm, chunk(c))
    accm = lax.fori_loop(0, n_full, max_body,
                         jnp.full((_BN, _C), -jnp.inf, jnp.float32),
                         unroll=2)
    m = jnp.max(accm, axis=-1, keepdims=True)
    if tail_w:
        m = jnp.maximum(m, jnp.max(x_ref[:, tail_st:], axis=-1, keepdims=True))

    tau2 = m - 2.0
    for _ in range(6):
        def newton_body(c, carry):
            acc1, acc2 = carry
            t = jnp.maximum(chunk(c) - tau2, 0.0)
            return acc1 + t, acc2 + t * t
        z = jnp.zeros((_BN, _C), jnp.float32)
        acc1, acc2 = lax.fori_loop(0, n_full, newton_body, (z, z), unroll=2)
        s1 = jnp.sum(acc1, axis=-1, keepdims=True)
        s2 = jnp.sum(acc2, axis=-1, keepdims=True)
        if tail_w:
            tt = jnp.maximum(x_ref[:, tail_st:] - tau2, 0.0)
            s1 = s1 + jnp.sum(tt, axis=-1, keepdims=True)
            s2 = s2 + jnp.sum(tt * tt, axis=-1, keepdims=True)
        tau2 = tau2 + (0.5 * s2 - 2.0) / (s1 + 1e-30)

    tgt = tgt_ref[...]
    base_col = lax.broadcasted_iota(jnp.int32, (_BN, _C), 1)

    def final_body(c, carry):
        acc_s, acc_sp, acc_d, acc_xt = carry
        xs = chunk(c)
        t = jnp.maximum(xs - tau2, 0.0)
        t2 = t * t
        hit = jnp.where(base_col + c * _C == tgt, xs, 0.0)
        return (acc_s + t2, acc_sp + t2 * t, acc_d + t2 * xs, acc_xt + hit)

    z = jnp.zeros((_BN, _C), jnp.float32)
    acc_s, acc_sp, acc_d, acc_xt = lax.fori_loop(
        0, n_full, final_body, (z, z, z, z), unroll=2)
    s2f = jnp.sum(acc_s, axis=-1, keepdims=True)
    s3f = jnp.sum(acc_sp, axis=-1, keepdims=True)
    df = jnp.sum(acc_d, axis=-1, keepdims=True)
    xt = jnp.sum(acc_xt, axis=-1, keepdims=True)
    if tail_w:
        xs = x_ref[:, tail_st:]
        t = jnp.maximum(xs - tau2, 0.0)
        t2 = t * t
        col = lax.broadcasted_iota(jnp.int32, (_BN, tail_w), 1)
        s2f = s2f + jnp.sum(t2, axis=-1, keepdims=True)
        s3f = s3f + jnp.sum(t2 * t, axis=-1, keepdims=True)
        df = df + jnp.sum(t2 * xs, axis=-1, keepdims=True)
        xt = xt + jnp.sum(jnp.where(col + tail_st == tgt, xs, 0.0),
                          axis=-1, keepdims=True)

    omega = (1.0 - s3f / (s2f * jnp.sqrt(s2f))) / 0.75
    out_ref[...] = omega + df / s2f - xt


def _row_losses(X, target2d):
    n, v = X.shape
    grid = n // _BN
    return pl.pallas_call(
        _loss_body,
        grid=(grid,),
        in_specs=[
            pl.BlockSpec((_BN, 1), lambda i: (i, 0)),
            pl.BlockSpec((_BN, v), lambda i: (i, 0)),
        ],
        out_specs=pl.BlockSpec((_BN, 1), lambda i: (i, 0)),
        out_shape=jax.ShapeDtypeStruct((n, 1), jnp.float32),
    )(target2d, X)


@jax.jit
def kernel(X, target):
    n = X.shape[0]
    losses = _row_losses(X, target.reshape(n, 1))
    return jnp.sum(losses) / float(n)


# speedup vs baseline: 2.1126x; 2.1126x over previous
import jax
import jax.numpy as jnp
from jax import lax
from jax.experimental import pallas as pl


_BN = 8
_C = 512


def _loss_body(tgt_ref, x_ref, out_ref):
    v = x_ref.shape[1]
    n_full = v // _C
    tail_st, tail_w = n_full * _C, v % _C

    bounds = [(c * _C, _C) for c in range(n_full)]
    if tail_w:
        bounds.append((tail_st, tail_w))

    parts = []
    for (st, w) in bounds:
        parts.append(jnp.max(x_ref[:, st:st + w], axis=-1, keepdims=True))
    m = parts[0]
    for p in parts[1:]:
        m = jnp.maximum(m, p)

    tau2 = m - 2.0
    for _ in range(6):
        s1_parts, s2_parts = [], []
        acc1 = jnp.zeros((_BN, _C), jnp.float32)
        acc2 = jnp.zeros((_BN, _C), jnp.float32)
        for (st, w) in bounds:
            t = jnp.maximum(x_ref[:, st:st + w] - tau2, 0.0)
            if w == _C:
                acc1 = acc1 + t
                acc2 = acc2 + t * t
            else:
                s1_parts.append(jnp.sum(t, axis=-1, keepdims=True))
                s2_parts.append(jnp.sum(t * t, axis=-1, keepdims=True))
        s1_parts.append(jnp.sum(acc1, axis=-1, keepdims=True))
        s2_parts.append(jnp.sum(acc2, axis=-1, keepdims=True))
        s1 = sum(s1_parts)
        s2 = sum(s2_parts)
        tau2 = tau2 + (0.5 * s2 - 2.0) / (s1 + 1e-30)

    tgt = tgt_ref[...]
    base_col = lax.broadcasted_iota(jnp.int32, (_BN, _C), 1)
    acc_s = jnp.zeros((_BN, _C), jnp.float32)
    acc_sp = jnp.zeros((_BN, _C), jnp.float32)
    acc_d = jnp.zeros((_BN, _C), jnp.float32)
    acc_xt = jnp.zeros((_BN, _C), jnp.float32)
    s_parts, sp_parts, d_parts, xt_parts = [], [], [], []
    for (st, w) in bounds:
        xs = x_ref[:, st:st + w]
        t = jnp.maximum(xs - tau2, 0.0)
        t2 = t * t
        if w == _C:
            hit = jnp.where(base_col + st == tgt, xs, 0.0)
            acc_s = acc_s + t2
            acc_sp = acc_sp + t2 * t
            acc_d = acc_d + t2 * xs
            acc_xt = acc_xt + hit
        else:
            col = lax.broadcasted_iota(jnp.int32, (_BN, w), 1)
            s_parts.append(jnp.sum(t2, axis=-1, keepdims=True))
            sp_parts.append(jnp.sum(t2 * t, axis=-1, keepdims=True))
            d_parts.append(jnp.sum(t2 * xs, axis=-1, keepdims=True))
            xt_parts.append(jnp.sum(jnp.where(col + st == tgt, xs, 0.0),
                                    axis=-1, keepdims=True))
    s_parts.append(jnp.sum(acc_s, axis=-1, keepdims=True))
    sp_parts.append(jnp.sum(acc_sp, axis=-1, keepdims=True))
    d_parts.append(jnp.sum(acc_d, axis=-1, keepdims=True))
    xt_parts.append(jnp.sum(acc_xt, axis=-1, keepdims=True))
    s2f = sum(s_parts)
    s3f = sum(sp_parts)
    df = sum(d_parts)
    xt = sum(xt_parts)

    omega = (1.0 - s3f / (s2f * jnp.sqrt(s2f))) / 0.75
    out_ref[...] = omega + df / s2f - xt


def _row_losses(X, target2d):
    n, v = X.shape
    grid = n // _BN
    return pl.pallas_call(
        _loss_body,
        grid=(grid,),
        in_specs=[
            pl.BlockSpec((_BN, 1), lambda i: (i, 0)),
            pl.BlockSpec((_BN, v), lambda i: (i, 0)),
        ],
        out_specs=pl.BlockSpec((_BN, 1), lambda i: (i, 0)),
        out_shape=jax.ShapeDtypeStruct((n, 1), jnp.float32),
    )(target2d, X)


@jax.jit
def kernel(X, target):
    n = X.shape[0]
    losses = _row_losses(X, target.reshape(n, 1))
    return jnp.sum(losses) / float(n)
